# baseline (speedup 1.0000x reference)
import functools

import jax
import jax.numpy as jnp
from jax import lax
from jax.experimental import pallas as pl
from jax.experimental.pallas import tpu as pltpu
from jax.experimental.pallas import tpu_sc as plsc

NC = 2
NS = 16
L = 16
NW = NC * NS

CH = 128


@functools.lru_cache(maxsize=None)
def _replay_kernel(buf_size: int, batch: int, q: int, d: int):
    tslice = ((buf_size + NS - 1) // NS + 2 * L - 1) // (2 * L) * (2 * L)
    npad = NS * tslice
    nchunks = batch // L
    sq = q // NW
    nk = sq // CH
    assert batch % L == 0 and q % NW == 0 and sq % CH == 0 and d % L == 0

    mesh = plsc.VectorSubcoreMesh(core_axis_name="c", subcore_axis_name="s")

    f32 = jnp.float32
    i32 = jnp.int32

    @functools.partial(
        pl.kernel,
        out_type=(
            jax.ShapeDtypeStruct((q, d), f32),
            jax.ShapeDtypeStruct((q,), i32),
            jax.ShapeDtypeStruct((q,), f32),
            jax.ShapeDtypeStruct((q, d), f32),
            jax.ShapeDtypeStruct((q,), f32),
            jax.ShapeDtypeStruct((NC * npad,), i32),
        ),
        mesh=mesh,
        compiler_params=pltpu.CompilerParams(needs_layout_passes=False,
                                             use_tc_tiling_on_sc=False),
        scratch_types=[
            pltpu.VMEM((tslice,), i32),
            pltpu.VMEM((batch,), i32),
            pltpu.VMEM((nk, CH), i32),
            pltpu.VMEM((nk, CH), i32),
            pltpu.VMEM((nk, CH), i32),
            pltpu.VMEM((nk, CH), i32),
            pltpu.VMEM((sq, d), f32),
            pltpu.VMEM((sq,), i32),
            pltpu.VMEM((sq,), i32),
            pltpu.VMEM((sq,), f32),
            pltpu.VMEM((sq,), f32),
            pltpu.VMEM((sq,), f32),
            pltpu.VMEM((sq,), f32),
            pltpu.SMEM((nk, CH), i32),
            pltpu.SMEM((nk, CH), i32),
            pltpu.VMEM_SHARED((NS, 2, nk, CH), i32),
            pltpu.SemaphoreType.DMA,
            pltpu.SemaphoreType.DMA,
        ],
    )
    def replay(widx_hbm, sidx_hbm, obs, nobs, act, rew, don,
               nu_obs, nu_nobs, nu_act, nu_rew, nu_don,
               o_obs, o_act, o_rew, o_nobs, o_don, table,
               tsl, widx_v, sidx_v, m_v, nidx_v, gidx_v, rows,
               act_old, act_new, rew_old, rew_new, don_old, don_new,
               sidx_s, m_s, hop, sem, rsem):
        cid = lax.axis_index("c")
        sid = lax.axis_index("s")
        wid = sid * NC + cid
        base = sid * tslice
        zero = jnp.zeros((L,), i32)

        def memset(i, carry):
            tsl[pl.ds(i * 2 * L, L)] = zero
            tsl[pl.ds(i * 2 * L + L, L)] = zero
            return carry

        lax.fori_loop(0, tslice // (2 * L), memset, 0)

        pltpu.sync_copy(widx_hbm, widx_v)

        iota = lax.iota(jnp.int32, L)

        def chunk(c, carry):
            idx = widx_v[pl.ds(c * L, L)]
            loc = idx - base
            m0 = (idx >= base) & (idx < base + tslice)
            vals = iota + (c * L + 1)

            def cond(carry_in):
                _, n = carry_in
                return n > 0

            def body(carry_in):
                m, _ = carry_in
                plsc.store_scatter(tsl, [loc], vals, mask=m)
                r = plsc.load_gather(tsl, [loc], mask=m)
                m2 = m & (r < vals)
                return m2, jnp.sum(jnp.where(m2, 1, 0))

            n0 = jnp.sum(jnp.where(m0, 1, 0))
            lax.while_loop(cond, body, (m0, n0))
            return carry

        lax.fori_loop(0, nchunks, chunk, 0)

        pltpu.sync_copy(tsl, table.at[pl.ds(cid * npad + base, tslice)])
        plsc.subcore_barrier()

        qbase = wid * sq

        for k in range(nk):
            pltpu.sync_copy(sidx_hbm.at[pl.ds(qbase + k * CH, CH)],
                            sidx_v.at[k])

        def mk_gidx(i, carry):
            k = i // (CH // L)
            s = (i % (CH // L)) * L
            gidx_v[k, pl.ds(s, L)] = sidx_v[k, pl.ds(s, L)] + cid * npad
            return carry

        lax.fori_loop(0, nk * (CH // L), mk_gidx, 0)

        descs = [pltpu.async_copy(table.at[gidx_v.at[k]], m_v.at[k], sem)
                 for k in range(nk)]
        for dsc in descs:
            dsc.wait()

        pltpu.sync_copy(sidx_v, hop.at[sid, 0])
        pltpu.sync_copy(m_v, hop.at[sid, 1])
        pltpu.sync_copy(hop.at[sid, 0], sidx_s)
        pltpu.sync_copy(hop.at[sid, 1], m_s)

        def mk_nidx(i, carry):
            k = i // (CH // L)
            s = (i % (CH // L)) * L
            mv = m_v[k, pl.ds(s, L)]
            nidx_v[k, pl.ds(s, L)] = jnp.maximum(mv - 1, 0)
            return carry

        lax.fori_loop(0, nk * (CH // L), mk_nidx, 0)

        descs = []
        for k in range(nk):
            descs.append(pltpu.async_copy(
                act.at[sidx_v.at[k]], act_old.at[pl.ds(k * CH, CH)], sem))
            descs.append(pltpu.async_copy(
                nu_act.at[nidx_v.at[k]], act_new.at[pl.ds(k * CH, CH)], sem))
            descs.append(pltpu.async_copy(
                rew.at[sidx_v.at[k]], rew_old.at[pl.ds(k * CH, CH)], sem))
            descs.append(pltpu.async_copy(
                nu_rew.at[nidx_v.at[k]], rew_new.at[pl.ds(k * CH, CH)], sem))
            descs.append(pltpu.async_copy(
                don.at[sidx_v.at[k]], don_old.at[pl.ds(k * CH, CH)], sem))
            descs.append(pltpu.async_copy(
                nu_don.at[nidx_v.at[k]], don_new.at[pl.ds(k * CH, CH)], sem))

        def make_row_fetch(src_old, src_new):
            def row_fetch(i, carry):
                k = i // CH
                c = i % CH
                mi = m_s[k, c]

                @pl.when(mi > 0)
                def _():
                    pltpu.async_copy(src_new.at[pl.ds(mi - 1, 1)],
                                     rows.at[pl.ds(i, 1)], rsem)

                @pl.when(mi <= 0)
                def _():
                    si = sidx_s[k, c]
                    pltpu.async_copy(src_old.at[pl.ds(si, 1)],
                                     rows.at[pl.ds(i, 1)], rsem)

                return carry
            return row_fetch

        lax.fori_loop(0, sq, make_row_fetch(obs, nu_obs), 0)

        for dsc in descs:
            dsc.wait()

        def merge_small(g, carry):
            k = g // (CH // L)
            s = (g % (CH // L)) * L
            m = m_v[k, pl.ds(s, L)] > 0

            @pl.when(jnp.any(m))
            def _():
                i_vec = g * L + iota
                for old_r, new_r in ((act_old, act_new), (rew_old, rew_new),
                                     (don_old, don_new)):
                    v = plsc.load_gather(new_r, [i_vec], mask=m)
                    plsc.store_scatter(old_r, [i_vec], v, mask=m)

            return carry

        lax.fori_loop(0, sq // L, merge_small, 0)

        pltpu.make_async_copy(obs.at[pl.ds(0, sq)], rows, rsem).wait()
        pltpu.sync_copy(rows, o_obs.at[pl.ds(qbase, sq)])

        lax.fori_loop(0, sq, make_row_fetch(nobs, nu_nobs), 0)
        pltpu.make_async_copy(nobs.at[pl.ds(0, sq)], rows, rsem).wait()
        pltpu.sync_copy(rows, o_nobs.at[pl.ds(qbase, sq)])

        pltpu.sync_copy(act_old, o_act.at[pl.ds(qbase, sq)])
        pltpu.sync_copy(rew_old, o_rew.at[pl.ds(qbase, sq)])
        pltpu.sync_copy(don_old, o_don.at[pl.ds(qbase, sq)])

    return replay


def kernel(obs, actions, rewards, next_obs, dones,
           new_obs, new_actions, new_rewards, new_next_obs, new_dones,
           write_idx, sample_idx):
    buf_size, d = obs.shape
    batch = write_idx.shape[0]
    q = sample_idx.shape[0]

    replay = _replay_kernel(buf_size, batch, q, d)
    out = replay(write_idx, sample_idx, obs, next_obs,
                 actions.reshape(buf_size), rewards.reshape(buf_size),
                 dones.reshape(buf_size),
                 new_obs, new_next_obs, new_actions.reshape(batch),
                 new_rewards.reshape(batch), new_dones.reshape(batch))
    return (out[0], out[1].reshape(q, 1), out[2].reshape(q, 1),
            out[3], out[4].reshape(q, 1))


# speedup vs baseline: 1.0472x; 1.0472x over previous
import functools

import jax
import jax.numpy as jnp
from jax import lax
from jax.experimental import pallas as pl
from jax.experimental.pallas import tpu as pltpu
from jax.experimental.pallas import tpu_sc as plsc

NC = 2
NS = 16
L = 16
NW = NC * NS

CH = 128


def _wid():
    return lax.axis_index("s") * NC + lax.axis_index("c")


@functools.lru_cache(maxsize=None)
def _build_table_kernel(buf_size: int, batch: int):
    tslice = ((buf_size + NW - 1) // NW + L - 1) // L * L
    npad = NW * tslice
    nchunks = batch // L
    assert batch % L == 0

    mesh = plsc.VectorSubcoreMesh(core_axis_name="c", subcore_axis_name="s")

    @functools.partial(
        pl.kernel,
        out_type=jax.ShapeDtypeStruct((npad,), jnp.int32),
        mesh=mesh,
        compiler_params=pltpu.CompilerParams(needs_layout_passes=False,
                                             use_tc_tiling_on_sc=False),
        scratch_types=[
            pltpu.VMEM((tslice,), jnp.int32),
            pltpu.VMEM((batch,), jnp.int32),
        ],
    )
    def build(widx_hbm, table_hbm, tsl, widx_v):
        wid = _wid()
        base = wid * tslice
        zero = jnp.zeros((L,), jnp.int32)

        def memset(i, carry):
            tsl[pl.ds(i * 2 * L, L)] = zero
            tsl[pl.ds(i * 2 * L + L, L)] = zero
            return carry

        lax.fori_loop(0, tslice // (2 * L), memset, 0)

        pltpu.sync_copy(widx_hbm, widx_v)

        iota = lax.iota(jnp.int32, L)

        def chunk(c, carry):
            idx = widx_v[pl.ds(c * L, L)]
            loc = idx - base
            m0 = (idx >= base) & (idx < base + tslice)
            vals = iota + (c * L + 1)

            def cond(carry_in):
                _, n = carry_in
                return n > 0

            def body(carry_in):
                m, _ = carry_in
                plsc.store_scatter(tsl, [loc], vals, mask=m)
                r = plsc.load_gather(tsl, [loc], mask=m)
                m2 = m & (r < vals)
                return m2, jnp.sum(jnp.where(m2, 1, 0))

            n0 = jnp.sum(jnp.where(m0, 1, 0))
            lax.while_loop(cond, body, (m0, n0))
            return carry

        lax.fori_loop(0, nchunks, chunk, 0)

        pltpu.sync_copy(tsl, table_hbm.at[pl.ds(base, tslice)])

    return build, npad


@functools.lru_cache(maxsize=None)
def _sample_kernel(buf_size: int, batch: int, q: int, d: int, npad: int):
    sq = q // NW
    assert q % NW == 0 and sq % CH == 0 and d % L == 0
    nk = sq // CH

    mesh = plsc.VectorSubcoreMesh(core_axis_name="c", subcore_axis_name="s")

    f32 = jnp.float32
    i32 = jnp.int32

    @functools.partial(
        pl.kernel,
        out_type=(
            jax.ShapeDtypeStruct((q, d), f32),
            jax.ShapeDtypeStruct((q,), i32),
            jax.ShapeDtypeStruct((q,), f32),
            jax.ShapeDtypeStruct((q, d), f32),
            jax.ShapeDtypeStruct((q,), f32),
        ),
        mesh=mesh,
        compiler_params=pltpu.CompilerParams(needs_layout_passes=False,
                                             use_tc_tiling_on_sc=False),
        scratch_types=[
            pltpu.VMEM((nk, CH), i32),
            pltpu.VMEM((nk, CH), i32),
            pltpu.VMEM((nk, CH), i32),
            pltpu.VMEM((sq, d), f32),
            pltpu.VMEM((sq, d), f32),
            pltpu.VMEM((sq,), i32),
            pltpu.VMEM((sq,), i32),
            pltpu.VMEM((sq,), f32),
            pltpu.VMEM((sq,), f32),
            pltpu.VMEM((sq,), f32),
            pltpu.VMEM((sq,), f32),
            pltpu.SMEM((nk, CH), i32),
            pltpu.SMEM((nk, CH), i32),
            pltpu.VMEM_SHARED((NS, 2, nk, CH), i32),
            pltpu.SemaphoreType.DMA,
            pltpu.SemaphoreType.DMA,
        ],
    )
    def sample(table, sidx_hbm, obs, nobs, act, rew, don,
               nu_obs, nu_nobs, nu_act, nu_rew, nu_don,
               o_obs, o_act, o_rew, o_nobs, o_don,
               sidx_v, m_v, nidx_v, rows_o, rows_n,
               act_old, act_new, rew_old, rew_new, don_old, don_new,
               sidx_s, m_s, hop, sem, rsem):
        wid = _wid()
        qbase = wid * sq

        for k in range(nk):
            pltpu.sync_copy(sidx_hbm.at[pl.ds(qbase + k * CH, CH)],
                            sidx_v.at[k])

        descs = [pltpu.async_copy(table.at[sidx_v.at[k]], m_v.at[k], sem)
                 for k in range(nk)]
        for dsc in descs:
            dsc.wait()

        sid = lax.axis_index("s")
        pltpu.sync_copy(sidx_v, hop.at[sid, 0])
        pltpu.sync_copy(m_v, hop.at[sid, 1])
        pltpu.sync_copy(hop.at[sid, 0], sidx_s)
        pltpu.sync_copy(hop.at[sid, 1], m_s)

        def mk_nidx(i, carry):
            k = i // (CH // L)
            s = (i % (CH // L)) * L
            mv = m_v[k, pl.ds(s, L)]
            nidx_v[k, pl.ds(s, L)] = jnp.maximum(mv - 1, 0)
            return carry

        lax.fori_loop(0, nk * (CH // L), mk_nidx, 0)

        descs = []
        for k in range(nk):
            descs.append(pltpu.async_copy(
                act.at[sidx_v.at[k]], act_old.at[pl.ds(k * CH, CH)], sem))
            descs.append(pltpu.async_copy(
                nu_act.at[nidx_v.at[k]], act_new.at[pl.ds(k * CH, CH)], sem))
            descs.append(pltpu.async_copy(
                rew.at[sidx_v.at[k]], rew_old.at[pl.ds(k * CH, CH)], sem))
            descs.append(pltpu.async_copy(
                nu_rew.at[nidx_v.at[k]], rew_new.at[pl.ds(k * CH, CH)], sem))
            descs.append(pltpu.async_copy(
                don.at[sidx_v.at[k]], don_old.at[pl.ds(k * CH, CH)], sem))
            descs.append(pltpu.async_copy(
                nu_don.at[nidx_v.at[k]], don_new.at[pl.ds(k * CH, CH)], sem))

        def row_fetch(i, carry):
            k = i // CH
            c = i % CH
            mi = m_s[k, c]

            @pl.when(mi > 0)
            def _():
                pltpu.async_copy(nu_obs.at[pl.ds(mi - 1, 1)],
                                 rows_o.at[pl.ds(i, 1)], rsem)
                pltpu.async_copy(nu_nobs.at[pl.ds(mi - 1, 1)],
                                 rows_n.at[pl.ds(i, 1)], rsem)

            @pl.when(mi <= 0)
            def _():
                si = sidx_s[k, c]
                pltpu.async_copy(obs.at[pl.ds(si, 1)],
                                 rows_o.at[pl.ds(i, 1)], rsem)
                pltpu.async_copy(nobs.at[pl.ds(si, 1)],
                                 rows_n.at[pl.ds(i, 1)], rsem)

            return carry

        lax.fori_loop(0, sq, row_fetch, 0)

        for dsc in descs:
            dsc.wait()

        iota = lax.iota(jnp.int32, L)

        def merge_small(g, carry):
            k = g // (CH // L)
            s = (g % (CH // L)) * L
            m = m_v[k, pl.ds(s, L)] > 0

            @pl.when(jnp.any(m))
            def _():
                i_vec = g * L + iota
                for old_r, new_r in ((act_old, act_new), (rew_old, rew_new),
                                     (don_old, don_new)):
                    v = plsc.load_gather(new_r, [i_vec], mask=m)
                    plsc.store_scatter(old_r, [i_vec], v, mask=m)

            return carry

        lax.fori_loop(0, sq // L, merge_small, 0)

        pltpu.make_async_copy(obs.at[pl.ds(0, sq)], rows_o, rsem).wait()
        pltpu.make_async_copy(nobs.at[pl.ds(0, sq)], rows_n, rsem).wait()

        pltpu.sync_copy(rows_o, o_obs.at[pl.ds(qbase, sq)])
        pltpu.sync_copy(rows_n, o_nobs.at[pl.ds(qbase, sq)])
        pltpu.sync_copy(act_old, o_act.at[pl.ds(qbase, sq)])
        pltpu.sync_copy(rew_old, o_rew.at[pl.ds(qbase, sq)])
        pltpu.sync_copy(don_old, o_don.at[pl.ds(qbase, sq)])

    return sample


def kernel(obs, actions, rewards, next_obs, dones,
           new_obs, new_actions, new_rewards, new_next_obs, new_dones,
           write_idx, sample_idx):
    buf_size, d = obs.shape
    batch = write_idx.shape[0]
    q = sample_idx.shape[0]

    build, npad = _build_table_kernel(buf_size, batch)
    table = build(write_idx)

    sample = _sample_kernel(buf_size, batch, q, d, npad)
    out = sample(table, sample_idx, obs, next_obs,
                 actions.reshape(buf_size), rewards.reshape(buf_size),
                 dones.reshape(buf_size),
                 new_obs, new_next_obs, new_actions.reshape(batch),
                 new_rewards.reshape(batch), new_dones.reshape(batch))
    return (out[0], out[1].reshape(q, 1), out[2].reshape(q, 1),
            out[3], out[4].reshape(q, 1))
